# baseline (speedup 1.0000x reference)
import functools

import jax
import jax.numpy as jnp
from jax import lax
from jax.experimental import pallas as pl
from jax.experimental.pallas import tpu as pltpu
from jax.experimental.pallas import tpu_sc as plsc

B, S = 4096, 200
D = 64
NTOT = B * S
NC, NS = 2, 16
NW = NC * NS
PER_W = NTOT // NW
CHUNK = 512
NSTEPS = PER_W // CHUNK


def _emb_body(x_hbm, table_hbm, out_hbm, idx_v, rows_v, gsem):
    wid = lax.axis_index("s") * NC + lax.axis_index("c")
    base = wid * PER_W

    @pl.loop(0, NSTEPS)
    def _step(i):
        off = base + i * CHUNK
        pltpu.sync_copy(x_hbm.at[pl.ds(off, CHUNK)], idx_v)
        pltpu.async_copy(table_hbm.at[idx_v], rows_v, gsem).wait()
        pltpu.sync_copy(rows_v, out_hbm.at[pl.ds(off, CHUNK)])


_emb = functools.partial(
    pl.kernel,
    out_type=jax.ShapeDtypeStruct((NTOT, D), jnp.float32),
    mesh=plsc.VectorSubcoreMesh(core_axis_name="c", subcore_axis_name="s"),
    scratch_types=[
        pltpu.VMEM((CHUNK,), jnp.int32),
        pltpu.VMEM((CHUNK, D), jnp.float32),
        pltpu.SemaphoreType.DMA,
    ],
    compiler_params=pltpu.CompilerParams(use_tc_tiling_on_sc=False),
)(_emb_body)


@jax.jit
def kernel(x, table):
    flat_idx = x.astype(jnp.int32).reshape(NTOT)
    out = _emb(flat_idx, table)
    return out.reshape(B, S, D)


# speedup vs baseline: 1.0396x; 1.0396x over previous
import functools

import jax
import jax.numpy as jnp
from jax import lax
from jax.experimental import pallas as pl
from jax.experimental.pallas import tpu as pltpu
from jax.experimental.pallas import tpu_sc as plsc

B, S = 4096, 200
D = 64
NTOT = B * S
NC, NS = 2, 16
NW = NC * NS
PER_W = NTOT // NW
CHUNK = 512
NSTEPS = PER_W // CHUNK
NBUF = 2
NGROUPS = NSTEPS // NBUF
assert NSTEPS % NBUF == 0


def _emb_body(x_hbm, table_hbm, out_hbm, idx_v, rows_v, isems, gsems, osems):
    wid = lax.axis_index("s") * NC + lax.axis_index("c")
    base = wid * PER_W

    def idx_copy(b, chunk):
        return pltpu.make_async_copy(
            x_hbm.at[pl.ds(base + chunk * CHUNK, CHUNK)], idx_v.at[b],
            isems.at[b])

    def gather_copy(b):
        return pltpu.make_async_copy(table_hbm.at[idx_v.at[b]], rows_v.at[b],
                                     gsems.at[b])

    def out_copy(b, chunk):
        return pltpu.make_async_copy(
            rows_v.at[b], out_hbm.at[pl.ds(base + chunk * CHUNK, CHUNK)],
            osems.at[b])

    for b in range(NBUF):
        idx_copy(b, b).start()
    for b in range(NBUF):
        idx_copy(b, b).wait()
        gather_copy(b).start()

    @pl.loop(NBUF, NSTEPS, step=NBUF)
    def _group(g):
        for b in range(NBUF):
            gather_copy(b).wait()
            idx_copy(b, g + b).start()
            out_copy(b, g - NBUF + b).start()
        for b in range(NBUF):
            out_copy(b, g - NBUF + b).wait()
            idx_copy(b, g + b).wait()
            gather_copy(b).start()

    for b in range(NBUF):
        gather_copy(b).wait()
        out_copy(b, NSTEPS - NBUF + b).start()
    for b in range(NBUF):
        out_copy(b, NSTEPS - NBUF + b).wait()


_emb = functools.partial(
    pl.kernel,
    out_type=jax.ShapeDtypeStruct((NTOT, D), jnp.float32),
    mesh=plsc.VectorSubcoreMesh(core_axis_name="c", subcore_axis_name="s"),
    scratch_types=[
        pltpu.VMEM((NBUF, CHUNK), jnp.int32),
        pltpu.VMEM((NBUF, CHUNK, D), jnp.float32),
        pltpu.SemaphoreType.DMA((NBUF,)),
        pltpu.SemaphoreType.DMA((NBUF,)),
        pltpu.SemaphoreType.DMA((NBUF,)),
    ],
    compiler_params=pltpu.CompilerParams(use_tc_tiling_on_sc=False),
)(_emb_body)


@jax.jit
def kernel(x, table):
    flat_idx = x.astype(jnp.int32).reshape(NTOT)
    out = _emb(flat_idx, table)
    return out.reshape(B, S, D)


# speedup vs baseline: 1.3838x; 1.3311x over previous
import functools

import jax
import jax.numpy as jnp
from jax import lax
from jax.experimental import pallas as pl
from jax.experimental.pallas import tpu as pltpu
from jax.experimental.pallas import tpu_sc as plsc
from jax.experimental.layout import Format, Layout, with_layout_constraint

B, S = 4096, 200
D = 64
NTOT = B * S
NC, NS = 2, 16
NW = NC * NS
PER_W = NTOT // NW
CHUNK = 512
NSTEPS = PER_W // CHUNK
NBUF = 2
NGROUPS = NSTEPS // NBUF
assert NSTEPS % NBUF == 0


def _emb_body(x_hbm, table_hbm, out_hbm, idx_v, rows_v, isems, gsems, osems):
    wid = lax.axis_index("s") * NC + lax.axis_index("c")
    base = wid * PER_W

    def idx_copy(b, chunk):
        return pltpu.make_async_copy(
            x_hbm.at[pl.ds(base + chunk * CHUNK, CHUNK)], idx_v.at[b],
            isems.at[b])

    def gather_copy(b):
        return pltpu.make_async_copy(table_hbm.at[idx_v.at[b]], rows_v.at[b],
                                     gsems.at[b])

    def out_copy(b, chunk):
        return pltpu.make_async_copy(
            rows_v.at[b],
            out_hbm.at[pl.ds(base + chunk * CHUNK, CHUNK), pl.ds(0, D)],
            osems.at[b])

    for b in range(NBUF):
        idx_copy(b, b).start()
    for b in range(NBUF):
        idx_copy(b, b).wait()
        gather_copy(b).start()

    @pl.loop(NBUF, NSTEPS, step=NBUF)
    def _group(g):
        for b in range(NBUF):
            gather_copy(b).wait()
            idx_copy(b, g + b).start()
            out_copy(b, g - NBUF + b).start()
        for b in range(NBUF):
            out_copy(b, g - NBUF + b).wait()
            idx_copy(b, g + b).wait()
            gather_copy(b).start()

    for b in range(NBUF):
        gather_copy(b).wait()
        out_copy(b, NSTEPS - NBUF + b).start()
    for b in range(NBUF):
        out_copy(b, NSTEPS - NBUF + b).wait()


_emb = functools.partial(
    pl.kernel,
    out_type=jax.ShapeDtypeStruct((NTOT, 2 * D), jnp.float32),
    mesh=plsc.VectorSubcoreMesh(core_axis_name="c", subcore_axis_name="s"),
    scratch_types=[
        pltpu.VMEM((NBUF, CHUNK), jnp.int32),
        pltpu.VMEM((NBUF, CHUNK, D), jnp.float32),
        pltpu.SemaphoreType.DMA((NBUF,)),
        pltpu.SemaphoreType.DMA((NBUF,)),
        pltpu.SemaphoreType.DMA((NBUF,)),
    ],
    compiler_params=pltpu.CompilerParams(use_tc_tiling_on_sc=False),
)(_emb_body)


@jax.jit
def kernel(x, table):
    flat_idx = x.astype(jnp.int32).reshape(NTOT)
    out = _emb(flat_idx, table)
    return out.reshape(B, S, 2 * D)[..., :D]


# speedup vs baseline: 1.3889x; 1.0037x over previous
import functools

import jax
import jax.numpy as jnp
from jax import lax
from jax.experimental import pallas as pl
from jax.experimental.pallas import tpu as pltpu
from jax.experimental.pallas import tpu_sc as plsc
from jax.experimental.layout import Format, Layout, with_layout_constraint

B, S = 4096, 200
D = 64
NTOT = B * S
NC, NS = 2, 16
NW = NC * NS
PER_W = NTOT // NW
CHUNK = 256
NSTEPS = PER_W // CHUNK
NBUF = 4
NGROUPS = NSTEPS // NBUF
assert NSTEPS % NBUF == 0


def _emb_body(x_hbm, table_hbm, out_hbm, idx_v, rows_v, isems, gsems, osems):
    wid = lax.axis_index("s") * NC + lax.axis_index("c")
    base = wid * PER_W

    def idx_copy(b, chunk):
        return pltpu.make_async_copy(
            x_hbm.at[pl.ds(base + chunk * CHUNK, CHUNK)], idx_v.at[b],
            isems.at[b])

    def gather_copy(b):
        return pltpu.make_async_copy(table_hbm.at[idx_v.at[b]], rows_v.at[b],
                                     gsems.at[b])

    def out_copy(b, chunk):
        return pltpu.make_async_copy(
            rows_v.at[b],
            out_hbm.at[pl.ds(base + chunk * CHUNK, CHUNK), pl.ds(0, D)],
            osems.at[b])

    for b in range(NBUF):
        idx_copy(b, b).start()
    for b in range(NBUF):
        idx_copy(b, b).wait()
        gather_copy(b).start()

    @pl.loop(NBUF, NSTEPS, step=NBUF)
    def _group(g):
        for b in range(NBUF):
            gather_copy(b).wait()
            idx_copy(b, g + b).start()
            out_copy(b, g - NBUF + b).start()
        for b in range(NBUF):
            out_copy(b, g - NBUF + b).wait()
            idx_copy(b, g + b).wait()
            gather_copy(b).start()

    for b in range(NBUF):
        gather_copy(b).wait()
        out_copy(b, NSTEPS - NBUF + b).start()
    for b in range(NBUF):
        out_copy(b, NSTEPS - NBUF + b).wait()


_emb = functools.partial(
    pl.kernel,
    out_type=jax.ShapeDtypeStruct((NTOT, 2 * D), jnp.float32),
    mesh=plsc.VectorSubcoreMesh(core_axis_name="c", subcore_axis_name="s"),
    scratch_types=[
        pltpu.VMEM((NBUF, CHUNK), jnp.int32),
        pltpu.VMEM((NBUF, CHUNK, D), jnp.float32),
        pltpu.SemaphoreType.DMA((NBUF,)),
        pltpu.SemaphoreType.DMA((NBUF,)),
        pltpu.SemaphoreType.DMA((NBUF,)),
    ],
    compiler_params=pltpu.CompilerParams(use_tc_tiling_on_sc=False),
)(_emb_body)


@jax.jit
def kernel(x, table):
    flat_idx = x.astype(jnp.int32).reshape(NTOT)
    out = _emb(flat_idx, table)
    return out.reshape(B, S, 2 * D)[..., :D]
